# Initial kernel scaffold; baseline (speedup 1.0000x reference)
import jax
import jax.numpy as jnp
from jax.experimental import pallas as pl


def kernel(x, gamma, beta, Wq, bq, Wk, bk, W1, b1, W2, b2):
    raise NotImplementedError("write your pallas kernel here")


# speedup vs baseline: 4.5982x; 4.5982x over previous
import functools

import jax
import jax.numpy as jnp
from jax import lax
from jax.experimental import pallas as pl
from jax.experimental.pallas import tpu as pltpu
from jax.experimental.pallas import tpu_sc as plsc

TEMP = 0.5


def _ln_body(x_ref, g_ref, b_ref, xn_ref, sums_ref):
    s = pl.program_id(1)
    xb = x_ref[0]
    mu = jnp.mean(xb, axis=1, keepdims=True)
    xc = xb - mu
    var = jnp.mean(xc * xc, axis=1, keepdims=True)
    xn = xc * lax.rsqrt(var + 1e-5) * g_ref[...] + b_ref[...]
    xn_ref[0] = xn

    @pl.when(s == 0)
    def _():
        sums_ref[...] = jnp.zeros_like(sums_ref)

    sums_ref[0] += jnp.sum(xn, axis=0, keepdims=True)


def _ln_call(x, gamma, beta):
    B, S, D = x.shape
    S_CHK = 256
    grid = (B, S // S_CHK)
    xn, sums = pl.pallas_call(
        _ln_body,
        grid=grid,
        in_specs=[
            pl.BlockSpec((1, S_CHK, D), lambda b, s: (b, s, 0)),
            pl.BlockSpec((1, D), lambda b, s: (0, 0)),
            pl.BlockSpec((1, D), lambda b, s: (0, 0)),
        ],
        out_specs=[
            pl.BlockSpec((1, S_CHK, D), lambda b, s: (b, s, 0)),
            pl.BlockSpec((1, 1, D), lambda b, s: (b, 0, 0)),
        ],
        out_shape=[
            jax.ShapeDtypeStruct((B, S, D), jnp.float32),
            jax.ShapeDtypeStruct((B, 1, D), jnp.float32),
        ],
    )(x, gamma.reshape(1, D), beta.reshape(1, D))
    return xn, sums


def _scores_body(sums_ref, wq_ref, bq_ref, wk_ref, bk_ref, sc_ref, *, B, T, S):
    repr_ = sums_ref[...] * (1.0 / S)
    q = jnp.dot(repr_, wq_ref[...], preferred_element_type=jnp.float32) + bq_ref[...]
    qn = jnp.sqrt(jnp.sum(q * q, axis=1, keepdims=True))
    q = q / jnp.maximum(qn, 1e-12)
    k = jnp.dot(repr_, wk_ref[...], preferred_element_type=jnp.float32) + bk_ref[...]
    kn = jnp.sqrt(jnp.sum(k * k, axis=1, keepdims=True))
    k = k / jnp.maximum(kn, 1e-12)
    rows = lax.broadcasted_iota(jnp.int32, (T, T), 0)
    cols = lax.broadcasted_iota(jnp.int32, (T, T), 1)
    diag = rows == cols
    for b in range(B):
        qb = q[b * T:(b + 1) * T]
        kb = k[b * T:(b + 1) * T]
        sc = lax.dot_general(qb, kb, (((1,), (1,)), ((), ())),
                             preferred_element_type=jnp.float32)
        sc_ref[b] = jnp.where(diag, -1e9, sc) * (1.0 / TEMP)


def _scores_call(sums, Wq, bq, Wk, bk, B, T, S):
    TD = Wq.shape[0]
    scores = pl.pallas_call(
        functools.partial(_scores_body, B=B, T=T, S=S),
        out_shape=jax.ShapeDtypeStruct((B, T, T), jnp.float32),
    )(sums.reshape(B * T, TD), Wq, bq.reshape(1, TD), Wk, bk.reshape(1, TD))
    return scores


def _routes_call(scores_flat):
    R, L = scores_flat.shape
    info = plsc.get_sparse_core_info()
    NC, NS = info.num_cores, info.num_subcores
    RPW = R // (NC * NS)
    mesh = plsc.VectorSubcoreMesh(core_axis_name="c", subcore_axis_name="s")

    @functools.partial(
        pl.kernel, mesh=mesh,
        compiler_params=pltpu.CompilerParams(needs_layout_passes=False),
        out_type=jax.ShapeDtypeStruct((R, L), jnp.int32),
        scratch_types=[
            pltpu.VMEM((RPW, L), jnp.float32),
            pltpu.VMEM((RPW, L), jnp.int32),
        ],
    )
    def k(sc_hbm, out_hbm, sc_v, idx_v):
        wid = lax.axis_index("s") * NC + lax.axis_index("c")
        base = wid * RPW
        pltpu.sync_copy(sc_hbm.at[pl.ds(base, RPW)], sc_v)
        iot = lax.iota(jnp.int32, L)
        for i in range(RPW):
            _, sv = plsc.sort_key_val(sc_v[i], iot, descending=True)
            idx_v[i] = sv
        pltpu.sync_copy(idx_v, out_hbm.at[pl.ds(base, RPW)])

    return k(scores_flat)


def _mlp_body(routes_ref, xs_ref, g0_ref, g1_ref, g2_ref, g3_ref, xres_ref,
              w1_ref, b1_ref, w2_ref, b2_ref, out_ref, *, TD):
    acc = jnp.dot(xs_ref[0], w1_ref[0:TD], preferred_element_type=jnp.float32)
    acc += jnp.dot(g0_ref[0], w1_ref[TD:2 * TD], preferred_element_type=jnp.float32)
    acc += jnp.dot(g1_ref[0], w1_ref[2 * TD:3 * TD], preferred_element_type=jnp.float32)
    acc += jnp.dot(g2_ref[0], w1_ref[3 * TD:4 * TD], preferred_element_type=jnp.float32)
    acc += jnp.dot(g3_ref[0], w1_ref[4 * TD:5 * TD], preferred_element_type=jnp.float32)
    z = acc + b1_ref[...]
    h = z * 0.5 * (1.0 + lax.erf(z * (2.0 ** -0.5)))
    out_ref[0] = (jnp.dot(h, w2_ref[...], preferred_element_type=jnp.float32)
                  + b2_ref[...] + xres_ref[0])


def _mlp_call(routes, x_norm, x, W1, b1, W2, b2):
    B, S, D = x.shape
    CTX, HID = W1.shape
    TD = W2.shape[1]
    T = D // TD
    S_CHK = 512
    grid = (B, T, S // S_CHK)

    def _self_idx(b, t, s, rt):
        return (b, s, t)

    def _gather_idx(kk):
        def idx(b, t, s, rt):
            return (b, s, rt[b, t, kk])
        return idx

    grid_spec = pltpu.PrefetchScalarGridSpec(
        num_scalar_prefetch=1,
        grid=grid,
        in_specs=[
            pl.BlockSpec((1, S_CHK, TD), _self_idx),
            pl.BlockSpec((1, S_CHK, TD), _gather_idx(0)),
            pl.BlockSpec((1, S_CHK, TD), _gather_idx(1)),
            pl.BlockSpec((1, S_CHK, TD), _gather_idx(2)),
            pl.BlockSpec((1, S_CHK, TD), _gather_idx(3)),
            pl.BlockSpec((1, S_CHK, TD), _self_idx),
            pl.BlockSpec((CTX, HID), lambda b, t, s, rt: (0, 0)),
            pl.BlockSpec((1, HID), lambda b, t, s, rt: (0, 0)),
            pl.BlockSpec((HID, TD), lambda b, t, s, rt: (0, 0)),
            pl.BlockSpec((1, TD), lambda b, t, s, rt: (0, 0)),
        ],
        out_specs=pl.BlockSpec((1, S_CHK, TD), _self_idx),
    )
    return pl.pallas_call(
        functools.partial(_mlp_body, TD=TD),
        grid_spec=grid_spec,
        out_shape=jax.ShapeDtypeStruct((B, S, D), jnp.float32),
    )(routes, x_norm, x_norm, x_norm, x_norm, x_norm, x,
      W1, b1.reshape(1, HID), W2, b2.reshape(1, TD))


def kernel(x, gamma, beta, Wq, bq, Wk, bk, W1, b1, W2, b2):
    B, S, D = x.shape
    TD = W2.shape[1]
    T = D // TD
    x_norm, sums = _ln_call(x, gamma, beta)
    scores = _scores_call(sums, Wq, bq, Wk, bk, B, T, S)
    idx_sorted = _routes_call(scores.reshape(B * T, T))
    routes = idx_sorted.reshape(B, T, T)[:, :, : (W1.shape[0] // TD - 1)]
    return _mlp_call(routes, x_norm, x, W1, b1, W2, b2)


# speedup vs baseline: 5.3214x; 1.1573x over previous
import functools

import jax
import jax.numpy as jnp
from jax import lax
from jax.experimental import pallas as pl
from jax.experimental.pallas import tpu as pltpu
from jax.experimental.pallas import tpu_sc as plsc

TEMP = 0.5


def _ln_body(x_ref, g_ref, b_ref, xn_ref, sums_ref):
    s = pl.program_id(1)
    xb = x_ref[0]
    mu = jnp.mean(xb, axis=1, keepdims=True)
    xc = xb - mu
    var = jnp.mean(xc * xc, axis=1, keepdims=True)
    xn = xc * lax.rsqrt(var + 1e-5) * g_ref[...] + b_ref[...]
    xn_ref[0] = xn.astype(jnp.bfloat16)

    @pl.when(s == 0)
    def _():
        sums_ref[...] = jnp.zeros_like(sums_ref)

    sums_ref[0] += jnp.sum(xn, axis=0, keepdims=True)


def _ln_call(x, gamma, beta):
    B, S, D = x.shape
    S_CHK = 256
    grid = (B, S // S_CHK)
    xn, sums = pl.pallas_call(
        _ln_body,
        grid=grid,
        in_specs=[
            pl.BlockSpec((1, S_CHK, D), lambda b, s: (b, s, 0)),
            pl.BlockSpec((1, D), lambda b, s: (0, 0)),
            pl.BlockSpec((1, D), lambda b, s: (0, 0)),
        ],
        out_specs=[
            pl.BlockSpec((1, S_CHK, D), lambda b, s: (b, s, 0)),
            pl.BlockSpec((1, 1, D), lambda b, s: (b, 0, 0)),
        ],
        out_shape=[
            jax.ShapeDtypeStruct((B, S, D), jnp.bfloat16),
            jax.ShapeDtypeStruct((B, 1, D), jnp.float32),
        ],
    )(x, gamma.reshape(1, D), beta.reshape(1, D))
    return xn, sums


def _scores_body(sums_ref, wq_ref, bq_ref, wk_ref, bk_ref, sc_ref, *, B, T, S):
    repr_ = sums_ref[...] * (1.0 / S)
    q = jnp.dot(repr_, wq_ref[...], preferred_element_type=jnp.float32) + bq_ref[...]
    qn = jnp.sqrt(jnp.sum(q * q, axis=1, keepdims=True))
    q = q / jnp.maximum(qn, 1e-12)
    k = jnp.dot(repr_, wk_ref[...], preferred_element_type=jnp.float32) + bk_ref[...]
    kn = jnp.sqrt(jnp.sum(k * k, axis=1, keepdims=True))
    k = k / jnp.maximum(kn, 1e-12)
    rows = lax.broadcasted_iota(jnp.int32, (T, T), 0)
    cols = lax.broadcasted_iota(jnp.int32, (T, T), 1)
    diag = rows == cols
    for b in range(B):
        qb = q[b * T:(b + 1) * T]
        kb = k[b * T:(b + 1) * T]
        sc = lax.dot_general(qb, kb, (((1,), (1,)), ((), ())),
                             preferred_element_type=jnp.float32)
        sc_ref[b] = jnp.where(diag, -1e9, sc) * (1.0 / TEMP)


def _scores_call(sums, Wq, bq, Wk, bk, B, T, S):
    TD = Wq.shape[0]
    scores = pl.pallas_call(
        functools.partial(_scores_body, B=B, T=T, S=S),
        out_shape=jax.ShapeDtypeStruct((B, T, T), jnp.float32),
    )(sums.reshape(B * T, TD), Wq, bq.reshape(1, TD), Wk, bk.reshape(1, TD))
    return scores


def _routes_call(scores_flat):
    R, L = scores_flat.shape
    info = plsc.get_sparse_core_info()
    NC, NS = info.num_cores, info.num_subcores
    RPW = R // (NC * NS)
    mesh = plsc.VectorSubcoreMesh(core_axis_name="c", subcore_axis_name="s")

    @functools.partial(
        pl.kernel, mesh=mesh,
        compiler_params=pltpu.CompilerParams(needs_layout_passes=False),
        out_type=jax.ShapeDtypeStruct((R, L), jnp.int32),
        scratch_types=[
            pltpu.VMEM((RPW, L), jnp.float32),
            pltpu.VMEM((RPW, L), jnp.int32),
        ],
    )
    def k(sc_hbm, out_hbm, sc_v, idx_v):
        wid = lax.axis_index("s") * NC + lax.axis_index("c")
        base = wid * RPW
        pltpu.sync_copy(sc_hbm.at[pl.ds(base, RPW)], sc_v)
        iot = lax.iota(jnp.int32, L)
        for i in range(RPW):
            _, sv = plsc.sort_key_val(sc_v[i], iot, descending=True)
            idx_v[i] = sv
        pltpu.sync_copy(idx_v, out_hbm.at[pl.ds(base, RPW)])

    return k(scores_flat)


def _mlp_body(routes_ref, xs_ref, g0_ref, g1_ref, g2_ref, g3_ref, xres_ref,
              w1_ref, b1_ref, w2_ref, b2_ref, out_ref, *, TD):
    acc = jnp.dot(xs_ref[0], w1_ref[0:TD], preferred_element_type=jnp.float32)
    acc += jnp.dot(g0_ref[0], w1_ref[TD:2 * TD], preferred_element_type=jnp.float32)
    acc += jnp.dot(g1_ref[0], w1_ref[2 * TD:3 * TD], preferred_element_type=jnp.float32)
    acc += jnp.dot(g2_ref[0], w1_ref[3 * TD:4 * TD], preferred_element_type=jnp.float32)
    acc += jnp.dot(g3_ref[0], w1_ref[4 * TD:5 * TD], preferred_element_type=jnp.float32)
    z = acc + b1_ref[...]
    h = z * 0.5 * (1.0 + lax.erf(z * (2.0 ** -0.5)))
    out_ref[0] = (jnp.dot(h.astype(jnp.bfloat16), w2_ref[...],
                          preferred_element_type=jnp.float32)
                  + b2_ref[...] + xres_ref[0])


def _mlp_call(routes, x_norm, x, W1, b1, W2, b2):
    B, S, D = x.shape
    CTX, HID = W1.shape
    TD = W2.shape[1]
    T = D // TD
    S_CHK = 512
    grid = (B, T, S // S_CHK)

    def _self_idx(b, t, s, rt):
        return (b, s, t)

    def _gather_idx(kk):
        def idx(b, t, s, rt):
            return (b, s, rt[b, t, kk])
        return idx

    grid_spec = pltpu.PrefetchScalarGridSpec(
        num_scalar_prefetch=1,
        grid=grid,
        in_specs=[
            pl.BlockSpec((1, S_CHK, TD), _self_idx),
            pl.BlockSpec((1, S_CHK, TD), _gather_idx(0)),
            pl.BlockSpec((1, S_CHK, TD), _gather_idx(1)),
            pl.BlockSpec((1, S_CHK, TD), _gather_idx(2)),
            pl.BlockSpec((1, S_CHK, TD), _gather_idx(3)),
            pl.BlockSpec((1, S_CHK, TD), _self_idx),
            pl.BlockSpec((CTX, HID), lambda b, t, s, rt: (0, 0)),
            pl.BlockSpec((1, HID), lambda b, t, s, rt: (0, 0)),
            pl.BlockSpec((HID, TD), lambda b, t, s, rt: (0, 0)),
            pl.BlockSpec((1, TD), lambda b, t, s, rt: (0, 0)),
        ],
        out_specs=pl.BlockSpec((1, S_CHK, TD), _self_idx),
    )
    return pl.pallas_call(
        functools.partial(_mlp_body, TD=TD),
        grid_spec=grid_spec,
        out_shape=jax.ShapeDtypeStruct((B, S, D), jnp.float32),
    )(routes, x_norm, x_norm, x_norm, x_norm, x_norm, x,
      W1.astype(jnp.bfloat16), b1.reshape(1, HID),
      W2.astype(jnp.bfloat16), b2.reshape(1, TD))


def kernel(x, gamma, beta, Wq, bq, Wk, bk, W1, b1, W2, b2):
    B, S, D = x.shape
    TD = W2.shape[1]
    T = D // TD
    x_norm, sums = _ln_call(x, gamma, beta)
    scores = _scores_call(sums, Wq, bq, Wk, bk, B, T, S)
    idx_sorted = _routes_call(scores.reshape(B * T, T))
    routes = idx_sorted.reshape(B, T, T)[:, :, : (W1.shape[0] // TD - 1)]
    return _mlp_call(routes, x_norm, x, W1, b1, W2, b2)
